# baseline (speedup 1.0000x reference)
import jax
import jax.numpy as jnp
from jax import lax
from jax.experimental import pallas as pl
from jax.experimental.pallas import tpu as pltpu
from jax.experimental.pallas import tpu_sc as plsc

NC = 2
NS = 16
L = 16


def _mesh():
  return plsc.VectorSubcoreMesh(
      core_axis_name="c", subcore_axis_name="s", num_cores=NC, num_subcores=NS)


def _scale_rows(gbuf, valbuf, n_edges, d_blocks):
  def _group(g, _):
    v16 = valbuf[pl.ds(g * L, L)]
    for k in range(L):
      e = g * L + k
      v = v16[k]
      for db in range(d_blocks):
        sl = pl.ds(db * L, L)
        gbuf[e, sl] = gbuf[e, sl] * v
    return 0
  lax.fori_loop(0, n_edges // L, _group, 0)


def _make_spmm_pair(E, G, D):
  ept = E // NS
  K = 160
  nch = ept // K
  assert nch * K == ept and (ept % 8) == 0
  r_hi = ((G // NS) + 7) // 8 * 8
  r_lo = G - (NS - 1) * r_hi
  assert r_lo > 0 and r_lo % 8 == 0

  def body(table, a3, z, cnt, out,
           acc, ebuf, gbuf2, idxb, cntbuf, semL, semG, semS):
    c = lax.axis_index("c")
    s = lax.axis_index("s")
    r0 = s * r_hi

    @pl.when(s < NS - 1)
    def _():
      pltpu.sync_copy(z.at[pl.ds(r0, r_hi)], acc.at[pl.ds(r0, r_hi)])

    @pl.when(s == NS - 1)
    def _():
      pltpu.sync_copy(z.at[pl.ds(r0, r_lo)], acc.at[pl.ds(r0, r_lo)])

    pltpu.sync_copy(cnt, cntbuf)
    n_ch = cntbuf[...][0].astype(jnp.int32)
    plsc.subcore_barrier()

    base3 = (c * NS + s) * nch * 3 * K

    def eslice(q):
      return a3.at[pl.ds(base3 + q * 3 * K, 3 * K)]

    pltpu.sync_copy(eslice(0), ebuf.at[pl.ds(0, 3 * K)])
    pltpu.async_copy(table.at[ebuf.at[pl.ds(K, K)]], gbuf2.at[0], semG)
    @pl.when(n_ch > 1)
    def _():
      pltpu.async_copy(eslice(1), ebuf.at[pl.ds(3 * K, 3 * K)], semL)

    def chunk(j, _):
      p = lax.rem(j, 2)
      eb = lax.rem(j, 3) * 3 * K
      pltpu.make_async_copy(table.at[ebuf.at[pl.ds(K, K)]],
                            gbuf2.at[p], semG).wait()
      @pl.when(j >= 1)
      def _():
        pltpu.make_async_copy(gbuf2.at[1 - p], acc.at[idxb.at[0, 0]],
                              semS).wait()
      @pl.when(j + 1 < n_ch)
      def _():
        eb1 = lax.rem(j + 1, 3) * 3 * K
        pltpu.make_async_copy(eslice(0), ebuf.at[pl.ds(eb1, 3 * K)],
                              semL).wait()
        pltpu.async_copy(table.at[ebuf.at[pl.ds(eb1 + K, K)]],
                         gbuf2.at[1 - p], semG)
      @pl.when(j + 2 < n_ch)
      def _():
        eb2 = lax.rem(j + 2, 3) * 3 * K
        pltpu.async_copy(eslice(j + 2), ebuf.at[pl.ds(eb2, 3 * K)], semL)
      def sgroup(g, _):
        v16 = plsc.bitcast(ebuf[pl.ds(eb + 2 * K + g * L, L)], jnp.float32)
        for k in range(L):
          e = g * L + k
          v = v16[k]
          for db in range(D // L):
            sl = pl.ds(db * L, L)
            gbuf2[p, e, sl] = gbuf2[p, e, sl] * v
        return 0
      lax.fori_loop(0, K // L, sgroup, 0)
      for i in range(K // L):
        idxb[p, 0, pl.ds(i * L, L)] = ebuf[pl.ds(eb + i * L, L)]
      pltpu.async_copy(gbuf2.at[p], acc.at[idxb.at[p, 0]], semS, add=True)
      return 0
    lax.fori_loop(0, n_ch, chunk, 0)
    pltpu.make_async_copy(gbuf2.at[0], acc.at[idxb.at[0, 0]], semS).wait()

    plsc.subcore_barrier()

    @pl.when(s < NS - 1)
    def _():
      pltpu.sync_copy(acc.at[pl.ds(r0, r_hi)], out.at[c, pl.ds(r0, r_hi)])

    @pl.when(s == NS - 1)
    def _():
      pltpu.sync_copy(acc.at[pl.ds(r0, r_lo)], out.at[c, pl.ds(r0, r_lo)])

  return pl.kernel(
      body,
      out_type=jax.ShapeDtypeStruct((NC, G, D), jnp.float32),
      mesh=_mesh(),
      compiler_params=pltpu.CompilerParams(needs_layout_passes=False),
      scratch_types=[
          pltpu.VMEM_SHARED((G, D), jnp.float32),
          pltpu.VMEM((9 * K,), jnp.int32),
          pltpu.VMEM((2, K, D), jnp.float32),
          pltpu.VMEM((2, 1, K), jnp.int32),
          pltpu.VMEM((L,), jnp.float32),
          pltpu.SemaphoreType.DMA,
          pltpu.SemaphoreType.DMA,
          pltpu.SemaphoreType.DMA,
      ],
  )


def _make_spmm_fine(G, Ny, W, D, Ep, P):
  ept = Ep // NS
  CHK = 768
  GK = 64
  FCAP = CHK + GK
  nch = ept // CHK
  assert nch * CHK == ept
  wr = W // NS
  assert wr * NS == W and wr % 8 == 0 and P * 2 * W == Ny

  def body(msg, e3, z, cnt, out,
           acc, ebuf, fltr, fltc, fltv, idxb, gbuf, cntbuf, sem, sem2):
    c = lax.axis_index("c")
    s = lax.axis_index("s")
    pltpu.sync_copy(cnt, cntbuf)
    n_ch = cntbuf[...][0].astype(jnp.int32)

    def pre_c(i, _):
      fltc[pl.ds(i * L, L)] = jnp.zeros((L,), jnp.int32)
      return 0
    lax.fori_loop(0, (FCAP + L) // L, pre_c, 0)

    def one_pass(p, _):
      lo = (2 * p + c) * W
      pltpu.sync_copy(z.at[pl.ds(0, wr)], acc.at[pl.ds(s * wr, wr)])
      @pl.when(s == 0)
      def _():
        pltpu.sync_copy(z.at[pl.ds(0, 8)], acc.at[pl.ds(W, 8)])
      plsc.subcore_barrier()

      base3 = s * nch * 3 * CHK
      pltpu.sync_copy(e3.at[pl.ds(base3, 3 * CHK)], ebuf.at[0])

      def chunk(j, _):
        pty = lax.rem(j, 2)
        npty = 1 - pty
        @pl.when(j + 1 < n_ch)
        def _():
          pltpu.async_copy(e3.at[pl.ds(base3 + (j + 1) * 3 * CHK, 3 * CHK)],
                           ebuf.at[npty], sem2)

        def fgroup(g, off):
          r = ebuf[pty, pl.ds(g * L, L)]
          m = (r >= lo) & (r < lo + W)
          cs = plsc.cumsum(m.astype(jnp.int32))
          lane = lax.iota(jnp.int32, L)
          idxv = jnp.where(m, off + cs - 1, FCAP + lane)
          plsc.store_scatter(fltr, [idxv], r - lo)
          plsc.store_scatter(fltc, [idxv], ebuf[pty, pl.ds(CHK + g * L, L)])
          plsc.store_scatter(fltv, [idxv],
                             plsc.bitcast(ebuf[pty, pl.ds(2 * CHK + g * L, L)],
                                          jnp.float32))
          return off + cs[15]
        off = lax.fori_loop(0, CHK // L, fgroup, jnp.int32(0))

        for i in range(GK // L):
          fltr[pl.ds(off + i * L, L)] = jnp.full((L,), W, jnp.int32)

        nb = lax.div(off + (GK - 1), GK)
        def drain(b, _):
          boff = b * GK
          for i in range(GK // L):
            idxb[0, pl.ds(i * L, L)] = fltr[pl.ds(boff + i * L, L)]
          pltpu.async_copy(msg.at[fltc.at[pl.ds(boff, GK)]], gbuf, sem).wait()
          _scale_rows(gbuf, fltv.at[pl.ds(boff, GK)], GK, D // L)
          pltpu.sync_copy(gbuf, acc.at[idxb.at[0]], add=True)
          return 0
        lax.fori_loop(0, nb, drain, 0)

        @pl.when(j + 1 < n_ch)
        def _():
          pltpu.make_async_copy(
              e3.at[pl.ds(base3 + (j + 1) * 3 * CHK, 3 * CHK)],
              ebuf.at[npty], sem2).wait()
        return 0
      lax.fori_loop(0, n_ch, chunk, 0)
      plsc.subcore_barrier()

      r0 = s * wr
      pltpu.sync_copy(acc.at[pl.ds(r0, wr)], out.at[pl.ds(lo + r0, wr), :])
      plsc.subcore_barrier()
      return 0

    lax.fori_loop(0, P, one_pass, 0)

  scratch = [
      pltpu.VMEM_SHARED((W + 8, D), jnp.float32),
      pltpu.VMEM((2, 3 * CHK), jnp.int32),
      pltpu.VMEM((FCAP + L,), jnp.int32),
      pltpu.VMEM((FCAP + L,), jnp.int32),
      pltpu.VMEM((FCAP + L,), jnp.float32),
      pltpu.VMEM((1, GK), jnp.int32),
      pltpu.VMEM((GK, D), jnp.float32),
      pltpu.VMEM((L,), jnp.float32),
      pltpu.SemaphoreType.DMA,
      pltpu.SemaphoreType.DMA,
  ]

  return pl.kernel(
      body,
      out_type=jax.ShapeDtypeStruct((Ny, D), jnp.float32),
      mesh=_mesh(),
      compiler_params=pltpu.CompilerParams(needs_layout_passes=False),
      scratch_types=scratch,
  )


def _make_tc_sum3(Ny, D):
  BR = 1024
  grid = Ny // BR
  assert grid * BR == Ny

  def body(a_r, b_r, c_r, o_r):
    o_r[...] = a_r[...] + b_r[...] + c_r[...]

  row_spec = pl.BlockSpec((BR, D), lambda i: (i, 0))
  return pl.pallas_call(
      body,
      grid=(grid,),
      in_specs=[row_spec, row_spec, row_spec],
      out_specs=row_spec,
      out_shape=jax.ShapeDtypeStruct((Ny, D), jnp.float32),
  )


def _make_tc_agg(G, D):
  BR = 1000
  grid = G // BR
  assert grid * BR == G

  def body(um_r, im_r, wa_r, wb_r, b_r, g_r, msg_r, go_r):
    m = (jnp.dot(um_r[...], wa_r[...], preferred_element_type=jnp.float32)
         + jnp.dot(im_r[...], wb_r[...], preferred_element_type=jnp.float32)
         + b_r[...])
    msg_r[...] = m
    go_r[...] = g_r[...] + m

  row_spec = pl.BlockSpec((BR, D), lambda i: (i, 0))
  full_spec = pl.BlockSpec((D, D), lambda i: (0, 0))
  return pl.pallas_call(
      body,
      grid=(grid,),
      in_specs=[row_spec, row_spec, full_spec, full_spec,
                pl.BlockSpec((1, D), lambda i: (0, 0)), row_spec],
      out_specs=[row_spec, row_spec],
      out_shape=[jax.ShapeDtypeStruct((G, D), jnp.float32),
                 jax.ShapeDtypeStruct((G, D), jnp.float32)],
  )


def kernel(user_emb, item_emb, group_emb, num_users, num_items,
           uh_rows, uh_cols, uh_vals, ih_rows, ih_cols, ih_vals,
           fh_rows, fh_cols, fh_vals, W0, b0, W1, b1):
  U, D = user_emb.shape
  I = item_emb.shape[0]
  G = group_emb.shape[0]
  N = U + I
  EU = uh_rows.shape[0]
  EF = fh_rows.shape[0]

  KA, KC = 160, 768
  W = 13824
  P = -(-N // (2 * W))
  Ny = P * 2 * W

  CHK_TOT = NS * KC
  Ep = ((EF + CHK_TOT - 1) // CHK_TOT) * CHK_TOT
  padn = Ep - EF
  fr = jnp.pad(fh_rows.astype(jnp.int32), (0, padn))
  fc = jnp.pad(fh_cols.astype(jnp.int32), (0, padn))
  fv = jnp.pad(fh_vals, (0, padn))
  fv_i = jax.lax.bitcast_convert_type(fv, jnp.int32)
  e3 = jnp.stack([fr.reshape(-1, KC), fc.reshape(-1, KC),
                  fv_i.reshape(-1, KC)], axis=1).reshape(-1)

  CHK_TOT_A = NS * KA
  Ea = ((EU + CHK_TOT_A - 1) // CHK_TOT_A) * CHK_TOT_A
  pada = Ea - EU
  rows2 = jnp.concatenate([
      jnp.pad(uh_rows.astype(jnp.int32), (0, pada)),
      jnp.pad(ih_rows.astype(jnp.int32), (0, pada))])
  cols2 = jnp.concatenate([
      jnp.pad(uh_cols.astype(jnp.int32), (0, pada)),
      jnp.pad(ih_cols.astype(jnp.int32) + U, (0, pada))])
  vals2 = jnp.concatenate([jnp.pad(uh_vals, (0, pada)),
                           jnp.pad(ih_vals, (0, pada))])
  vals2_i = jax.lax.bitcast_convert_type(vals2, jnp.int32)
  a3 = jnp.stack([rows2.reshape(-1, KA), cols2.reshape(-1, KA),
                  vals2_i.reshape(-1, KA)], axis=1).reshape(-1)

  zeros_g = jnp.zeros((G, D), jnp.float32)
  cnt_a = jnp.full((L,), (Ea // NS) // KA, jnp.float32)
  cnt_c = jnp.full((L,), (Ep // NS) // KC, jnp.float32)

  w0a = W0[:, :D].T
  w0b = W0[:, D:].T
  w1a = W1[:, :D].T
  w1b = W1[:, D:].T
  b0r = b0.reshape(1, D)
  b1r = b1.reshape(1, D)

  spmm_a = _make_spmm_pair(Ea, G, D)
  spmm_c = _make_spmm_fine(G, Ny, W, D, Ep, P)
  tc_agg = _make_tc_agg(G, D)
  tc_sum3 = _make_tc_sum3(Ny, D)

  tab1 = jnp.concatenate([user_emb, item_emb], axis=0)
  pair1 = spmm_a(tab1, a3, zeros_g, cnt_a)
  msg1, g1 = tc_agg(pair1[0], pair1[1], w0a, w0b, b0r, group_emb)
  norm1 = spmm_c(msg1, e3, zeros_g, cnt_c)

  pair2 = spmm_a(norm1, a3, zeros_g, cnt_a)
  msg2, g_out = tc_agg(pair2[0], pair2[1], w1a, w1b, b1r, g1)
  norm2 = spmm_c(msg2, e3, zeros_g, cnt_c)

  ui0p = jnp.concatenate(
      [user_emb, item_emb, jnp.zeros((Ny - N, D), jnp.float32)])
  ui_out = tc_sum3(ui0p, norm1, norm2)[:N]

  return (ui_out, g_out)


# speedup vs baseline: 1.1546x; 1.1546x over previous
import jax
import jax.numpy as jnp
from jax import lax
from jax.experimental import pallas as pl
from jax.experimental.pallas import tpu as pltpu
from jax.experimental.pallas import tpu_sc as plsc

NC = 2
NS = 16
L = 16


def _mesh():
  return plsc.VectorSubcoreMesh(
      core_axis_name="c", subcore_axis_name="s", num_cores=NC, num_subcores=NS)


def _scale_rows(gbuf, valbuf, n_edges, d_blocks):
  def _group(g, _):
    v16 = valbuf[pl.ds(g * L, L)]
    for k in range(L):
      e = g * L + k
      v = v16[k]
      for db in range(d_blocks):
        sl = pl.ds(db * L, L)
        gbuf[e, sl] = gbuf[e, sl] * v
    return 0
  lax.fori_loop(0, n_edges // L, _group, 0)


def _make_spmm_pair(E, G, D):
  ept = E // NS
  K = 320
  nch = ept // K
  assert nch * K == ept and (ept % 8) == 0
  r_hi = ((G // NS) + 7) // 8 * 8
  r_lo = G - (NS - 1) * r_hi
  assert r_lo > 0 and r_lo % 8 == 0

  def body(table, a3, z, cnt, out,
           acc, ebuf, gbuf, idxb, cntbuf, sem, semL):
    c = lax.axis_index("c")
    s = lax.axis_index("s")
    r0 = s * r_hi

    @pl.when(s < NS - 1)
    def _():
      pltpu.sync_copy(z.at[pl.ds(r0, r_hi)], acc.at[pl.ds(r0, r_hi)])

    @pl.when(s == NS - 1)
    def _():
      pltpu.sync_copy(z.at[pl.ds(r0, r_lo)], acc.at[pl.ds(r0, r_lo)])

    pltpu.sync_copy(cnt, cntbuf)
    n_ch = cntbuf[...][0].astype(jnp.int32)
    plsc.subcore_barrier()

    base3 = (c * NS + s) * nch * 3 * K
    pltpu.sync_copy(a3.at[pl.ds(base3, 3 * K)], ebuf.at[pl.ds(0, 3 * K)])

    def chunk(j, _):
      eb = lax.rem(j, 2) * 3 * K
      @pl.when(j + 1 < n_ch)
      def _():
        neb = lax.rem(j + 1, 2) * 3 * K
        pltpu.async_copy(a3.at[pl.ds(base3 + (j + 1) * 3 * K, 3 * K)],
                         ebuf.at[pl.ds(neb, 3 * K)], semL)
      pltpu.async_copy(table.at[ebuf.at[pl.ds(eb + K, K)]], gbuf, sem).wait()
      def sgroup(g, _):
        v16 = plsc.bitcast(ebuf[pl.ds(eb + 2 * K + g * L, L)], jnp.float32)
        for k in range(L):
          e = g * L + k
          v = v16[k]
          for db in range(D // L):
            sl = pl.ds(db * L, L)
            gbuf[e, sl] = gbuf[e, sl] * v
        return 0
      lax.fori_loop(0, K // L, sgroup, 0)
      for i in range(K // L):
        idxb[0, pl.ds(i * L, L)] = ebuf[pl.ds(eb + i * L, L)]
      pltpu.sync_copy(gbuf, acc.at[idxb.at[0]], add=True)
      @pl.when(j + 1 < n_ch)
      def _():
        neb = lax.rem(j + 1, 2) * 3 * K
        pltpu.make_async_copy(a3.at[pl.ds(base3 + (j + 1) * 3 * K, 3 * K)],
                              ebuf.at[pl.ds(neb, 3 * K)], semL).wait()
      return 0
    lax.fori_loop(0, n_ch, chunk, 0)

    plsc.subcore_barrier()

    @pl.when(s < NS - 1)
    def _():
      pltpu.sync_copy(acc.at[pl.ds(r0, r_hi)], out.at[c, pl.ds(r0, r_hi)])

    @pl.when(s == NS - 1)
    def _():
      pltpu.sync_copy(acc.at[pl.ds(r0, r_lo)], out.at[c, pl.ds(r0, r_lo)])

  return pl.kernel(
      body,
      out_type=jax.ShapeDtypeStruct((NC, G, D), jnp.float32),
      mesh=_mesh(),
      compiler_params=pltpu.CompilerParams(needs_layout_passes=False),
      scratch_types=[
          pltpu.VMEM_SHARED((G, D), jnp.float32),
          pltpu.VMEM((6 * K,), jnp.int32),
          pltpu.VMEM((K, D), jnp.float32),
          pltpu.VMEM((1, K), jnp.int32),
          pltpu.VMEM((L,), jnp.float32),
          pltpu.SemaphoreType.DMA,
          pltpu.SemaphoreType.DMA,
      ],
  )


def _make_spmm_fine(G, Ny, W, D, Ep, P):
  ept = Ep // NS
  CHK = 768
  GK = 64
  FCAP = CHK + GK
  nch = ept // CHK
  assert nch * CHK == ept
  wr = W // NS
  assert wr * NS == W and wr % 8 == 0 and P * 2 * W == Ny

  def body(msg, e3, z, cnt, out,
           acc, ebuf, fltr, fltc, fltv, idxb, gbuf, cntbuf, sem, sem2):
    c = lax.axis_index("c")
    s = lax.axis_index("s")
    pltpu.sync_copy(cnt, cntbuf)
    n_ch = cntbuf[...][0].astype(jnp.int32)

    def pre_c(i, _):
      fltc[pl.ds(i * L, L)] = jnp.zeros((L,), jnp.int32)
      return 0
    lax.fori_loop(0, (FCAP + L) // L, pre_c, 0)

    def one_pass(p, _):
      lo = (2 * p + c) * W
      pltpu.sync_copy(z.at[pl.ds(0, wr)], acc.at[pl.ds(s * wr, wr)])
      @pl.when(s == 0)
      def _():
        pltpu.sync_copy(z.at[pl.ds(0, 8)], acc.at[pl.ds(W, 8)])
      plsc.subcore_barrier()

      base3 = s * nch * 3 * CHK
      pltpu.sync_copy(e3.at[pl.ds(base3, 3 * CHK)], ebuf.at[0])

      def chunk(j, _):
        pty = lax.rem(j, 2)
        npty = 1 - pty
        @pl.when(j + 1 < n_ch)
        def _():
          pltpu.async_copy(e3.at[pl.ds(base3 + (j + 1) * 3 * CHK, 3 * CHK)],
                           ebuf.at[npty], sem2)

        def fgroup(g, off):
          r = ebuf[pty, pl.ds(g * L, L)]
          m = (r >= lo) & (r < lo + W)
          cs = plsc.cumsum(m.astype(jnp.int32))
          lane = lax.iota(jnp.int32, L)
          idxv = jnp.where(m, off + cs - 1, FCAP + lane)
          plsc.store_scatter(fltr, [idxv], r - lo)
          plsc.store_scatter(fltc, [idxv], ebuf[pty, pl.ds(CHK + g * L, L)])
          plsc.store_scatter(fltv, [idxv],
                             plsc.bitcast(ebuf[pty, pl.ds(2 * CHK + g * L, L)],
                                          jnp.float32))
          return off + cs[15]
        off = lax.fori_loop(0, CHK // L, fgroup, jnp.int32(0))

        for i in range(GK // L):
          fltr[pl.ds(off + i * L, L)] = jnp.full((L,), W, jnp.int32)

        nb = lax.div(off + (GK - 1), GK)
        def drain(b, _):
          boff = b * GK
          for i in range(GK // L):
            idxb[0, pl.ds(i * L, L)] = fltr[pl.ds(boff + i * L, L)]
          pltpu.async_copy(msg.at[fltc.at[pl.ds(boff, GK)]], gbuf, sem).wait()
          _scale_rows(gbuf, fltv.at[pl.ds(boff, GK)], GK, D // L)
          pltpu.sync_copy(gbuf, acc.at[idxb.at[0]], add=True)
          return 0
        lax.fori_loop(0, nb, drain, 0)

        @pl.when(j + 1 < n_ch)
        def _():
          pltpu.make_async_copy(
              e3.at[pl.ds(base3 + (j + 1) * 3 * CHK, 3 * CHK)],
              ebuf.at[npty], sem2).wait()
        return 0
      lax.fori_loop(0, n_ch, chunk, 0)
      plsc.subcore_barrier()

      r0 = s * wr
      pltpu.sync_copy(acc.at[pl.ds(r0, wr)], out.at[pl.ds(lo + r0, wr), :])
      plsc.subcore_barrier()
      return 0

    lax.fori_loop(0, P, one_pass, 0)

  scratch = [
      pltpu.VMEM_SHARED((W + 8, D), jnp.float32),
      pltpu.VMEM((2, 3 * CHK), jnp.int32),
      pltpu.VMEM((FCAP + L,), jnp.int32),
      pltpu.VMEM((FCAP + L,), jnp.int32),
      pltpu.VMEM((FCAP + L,), jnp.float32),
      pltpu.VMEM((1, GK), jnp.int32),
      pltpu.VMEM((GK, D), jnp.float32),
      pltpu.VMEM((L,), jnp.float32),
      pltpu.SemaphoreType.DMA,
      pltpu.SemaphoreType.DMA,
  ]

  return pl.kernel(
      body,
      out_type=jax.ShapeDtypeStruct((Ny, D), jnp.float32),
      mesh=_mesh(),
      compiler_params=pltpu.CompilerParams(needs_layout_passes=False),
      scratch_types=scratch,
  )


def _make_tc_sum3(Ny, D):
  BR = 1024
  grid = Ny // BR
  assert grid * BR == Ny

  def body(a_r, b_r, c_r, o_r):
    o_r[...] = a_r[...] + b_r[...] + c_r[...]

  row_spec = pl.BlockSpec((BR, D), lambda i: (i, 0))
  return pl.pallas_call(
      body,
      grid=(grid,),
      in_specs=[row_spec, row_spec, row_spec],
      out_specs=row_spec,
      out_shape=jax.ShapeDtypeStruct((Ny, D), jnp.float32),
  )


def _make_tc_agg(G, D):
  BR = 1000
  grid = G // BR
  assert grid * BR == G

  def body(um_r, im_r, wa_r, wb_r, b_r, g_r, msg_r, go_r):
    m = (jnp.dot(um_r[...], wa_r[...], preferred_element_type=jnp.float32)
         + jnp.dot(im_r[...], wb_r[...], preferred_element_type=jnp.float32)
         + b_r[...])
    msg_r[...] = m
    go_r[...] = g_r[...] + m

  row_spec = pl.BlockSpec((BR, D), lambda i: (i, 0))
  full_spec = pl.BlockSpec((D, D), lambda i: (0, 0))
  return pl.pallas_call(
      body,
      grid=(grid,),
      in_specs=[row_spec, row_spec, full_spec, full_spec,
                pl.BlockSpec((1, D), lambda i: (0, 0)), row_spec],
      out_specs=[row_spec, row_spec],
      out_shape=[jax.ShapeDtypeStruct((G, D), jnp.float32),
                 jax.ShapeDtypeStruct((G, D), jnp.float32)],
  )


def kernel(user_emb, item_emb, group_emb, num_users, num_items,
           uh_rows, uh_cols, uh_vals, ih_rows, ih_cols, ih_vals,
           fh_rows, fh_cols, fh_vals, W0, b0, W1, b1):
  U, D = user_emb.shape
  I = item_emb.shape[0]
  G = group_emb.shape[0]
  N = U + I
  EU = uh_rows.shape[0]
  EF = fh_rows.shape[0]

  KA, KC = 320, 768
  W = 13824
  P = -(-N // (2 * W))
  Ny = P * 2 * W

  CHK_TOT = NS * KC
  Ep = ((EF + CHK_TOT - 1) // CHK_TOT) * CHK_TOT
  padn = Ep - EF
  fr = jnp.pad(fh_rows.astype(jnp.int32), (0, padn))
  fc = jnp.pad(fh_cols.astype(jnp.int32), (0, padn))
  fv = jnp.pad(fh_vals, (0, padn))
  fv_i = jax.lax.bitcast_convert_type(fv, jnp.int32)
  e3 = jnp.stack([fr.reshape(-1, KC), fc.reshape(-1, KC),
                  fv_i.reshape(-1, KC)], axis=1).reshape(-1)

  CHK_TOT_A = NS * KA
  Ea = ((EU + CHK_TOT_A - 1) // CHK_TOT_A) * CHK_TOT_A
  pada = Ea - EU
  rows2 = jnp.concatenate([
      jnp.pad(uh_rows.astype(jnp.int32), (0, pada)),
      jnp.pad(ih_rows.astype(jnp.int32), (0, pada))])
  cols2 = jnp.concatenate([
      jnp.pad(uh_cols.astype(jnp.int32), (0, pada)),
      jnp.pad(ih_cols.astype(jnp.int32) + U, (0, pada))])
  vals2 = jnp.concatenate([jnp.pad(uh_vals, (0, pada)),
                           jnp.pad(ih_vals, (0, pada))])
  vals2_i = jax.lax.bitcast_convert_type(vals2, jnp.int32)
  a3 = jnp.stack([rows2.reshape(-1, KA), cols2.reshape(-1, KA),
                  vals2_i.reshape(-1, KA)], axis=1).reshape(-1)

  zeros_g = jnp.zeros((G, D), jnp.float32)
  cnt_a = jnp.full((L,), (Ea // NS) // KA, jnp.float32)
  cnt_c = jnp.full((L,), (Ep // NS) // KC, jnp.float32)

  w0a = W0[:, :D].T
  w0b = W0[:, D:].T
  w1a = W1[:, :D].T
  w1b = W1[:, D:].T
  b0r = b0.reshape(1, D)
  b1r = b1.reshape(1, D)

  spmm_a = _make_spmm_pair(Ea, G, D)
  spmm_c = _make_spmm_fine(G, Ny, W, D, Ep, P)
  tc_agg = _make_tc_agg(G, D)
  tc_sum3 = _make_tc_sum3(Ny, D)

  tab1 = jnp.concatenate([user_emb, item_emb], axis=0)
  pair1 = spmm_a(tab1, a3, zeros_g, cnt_a)
  msg1, g1 = tc_agg(pair1[0], pair1[1], w0a, w0b, b0r, group_emb)
  norm1 = spmm_c(msg1, e3, zeros_g, cnt_c)

  pair2 = spmm_a(norm1, a3, zeros_g, cnt_a)
  msg2, g_out = tc_agg(pair2[0], pair2[1], w1a, w1b, b1r, g1)
  norm2 = spmm_c(msg2, e3, zeros_g, cnt_c)

  ui0p = jnp.concatenate(
      [user_emb, item_emb, jnp.zeros((Ny - N, D), jnp.float32)])
  ui_out = tc_sum3(ui0p, norm1, norm2)[:N]

  return (ui_out, g_out)
